# Initial kernel scaffold; baseline (speedup 1.0000x reference)
import jax
import jax.numpy as jnp
from jax.experimental import pallas as pl


def kernel(z, embedding):
    raise NotImplementedError("write your pallas kernel here")


# speedup vs baseline: 1.0153x; 1.0153x over previous
import functools

import jax
import jax.numpy as jnp
from jax import lax
from jax.experimental import pallas as pl
from jax.experimental.pallas import tpu as pltpu
from jax.experimental.pallas import tpu_sc as plsc

_N_E = 8192
_E_DIM = 256
_BETA = 0.1

_TM = 512
_TS = 4096


def _dist_argmin_body(zf_ref, embt_ref, zn_ref, en_ref, inds_ref, dmin_ref):
    zf = zf_ref[...]
    zn = zn_ref[0]
    cols = lax.broadcasted_iota(jnp.int32, (_TM, _TS), 1)
    mins, args = [], []
    for jj in range(_N_E // _TS):
        s = jnp.dot(zf, embt_ref[:, jj * _TS:(jj + 1) * _TS],
                    preferred_element_type=jnp.float32)
        en = en_ref[:, jj * _TS:(jj + 1) * _TS]
        d = (zn + en) - 2.0 * s
        lm = jnp.min(d, axis=1)
        la = jnp.min(jnp.where(d == lm[:, None], cols, jnp.int32(2**30)),
                     axis=1) + jj * _TS
        mins.append(lm)
        args.append(la)
    v0b = mins[0].astype(jnp.bfloat16).astype(jnp.float32)
    upd = mins[1] < v0b
    bidx = jnp.where(upd, args[1], args[0])
    bval = jnp.where(upd, mins[1], mins[0])
    inds_ref[0] = bidx.reshape(1, _TM)
    dmin_ref[0] = bval.reshape(1, _TM)


def _dist_argmin(zf, embt, zn3, en2, bl):
    nbm = bl // _TM
    return pl.pallas_call(
        _dist_argmin_body,
        grid=(nbm,),
        in_specs=[
            pl.BlockSpec((_TM, _E_DIM), lambda i: (i, 0)),
            pl.BlockSpec((_E_DIM, _N_E), lambda i: (0, 0)),
            pl.BlockSpec((1, _TM, 1), lambda i: (i, 0, 0)),
            pl.BlockSpec((1, _N_E), lambda i: (0, 0)),
        ],
        out_specs=[
            pl.BlockSpec((1, 1, _TM), lambda i: (i, 0, 0)),
            pl.BlockSpec((1, 1, _TM), lambda i: (i, 0, 0)),
        ],
        out_shape=[
            jax.ShapeDtypeStruct((nbm, 1, _TM), jnp.int32),
            jax.ShapeDtypeStruct((nbm, 1, _TM), jnp.float32),
        ],
    )(zf, embt, zn3, en2)


def _sc_gather(embedding, inds, bl):
    info = plsc.get_sparse_core_info()
    nw = info.num_cores * info.num_subcores
    b_per_w = bl // nw
    mesh = plsc.VectorSubcoreMesh(core_axis_name="c", subcore_axis_name="s")

    @functools.partial(
        pl.kernel,
        out_type=jax.ShapeDtypeStruct((bl, _E_DIM), jnp.float32),
        mesh=mesh,
        scratch_types=[
            pltpu.VMEM((b_per_w,), jnp.int32),
            pltpu.VMEM((b_per_w, _E_DIM), jnp.float32),
            pltpu.SemaphoreType.DMA,
        ],
    )
    def gather(emb_hbm, idx_hbm, out_hbm, idx_v, rows_v, sem):
        wid = lax.axis_index("s") * info.num_cores + lax.axis_index("c")
        base = wid * b_per_w
        pltpu.sync_copy(idx_hbm.at[pl.ds(base, b_per_w)], idx_v)
        pltpu.async_copy(emb_hbm.at[idx_v], rows_v, sem).wait()
        pltpu.sync_copy(rows_v, out_hbm.at[pl.ds(base, b_per_w)])

    return gather(embedding, inds)


def kernel(z, embedding):
    B, C, L = z.shape
    bl = B * L
    zp = jnp.transpose(z, (0, 2, 1))
    zf = zp.reshape(-1, C)
    zn = jnp.sum(zf ** 2, axis=1)
    en = jnp.sum(embedding ** 2, axis=1)
    embt = embedding.T

    inds3, dmin3 = _dist_argmin(
        zf, embt, zn.reshape(bl // _TM, _TM, 1), en.reshape(1, _N_E), bl)
    inds = inds3.reshape(-1)
    dmin = dmin3.reshape(-1)

    zq_rows = jnp.take(embedding, inds, axis=0)

    m = jnp.sum(dmin) / (B * C * L)
    loss = _BETA * m + m

    zq = zq_rows.reshape(B, L, C)
    z_q = zp + (zq - zp)
    z_q = jnp.transpose(z_q, (0, 2, 1))
    return (z_q, loss, inds)


# speedup vs baseline: 1.0306x; 1.0150x over previous
import functools

import jax
import jax.numpy as jnp
from jax import lax
from jax.experimental import pallas as pl
from jax.experimental.pallas import tpu as pltpu
from jax.experimental.pallas import tpu_sc as plsc

_N_E = 8192
_E_DIM = 256
_BETA = 0.1

_TM = 512
_TS = 4096


def _dist_argmin_body(zf_ref, embt_ref, zn_ref, en_ref, inds_ref, dmin_ref):
    zf = zf_ref[...]
    zn = zn_ref[0]
    cols = lax.broadcasted_iota(jnp.int32, (_TM, _TS), 1)
    mins, args = [], []
    for jj in range(_N_E // _TS):
        s = jnp.dot(zf, embt_ref[:, jj * _TS:(jj + 1) * _TS],
                    preferred_element_type=jnp.float32)
        en = en_ref[:, jj * _TS:(jj + 1) * _TS]
        d = (zn + en) - 2.0 * s
        lm = jnp.min(d, axis=1)
        la = jnp.min(jnp.where(d == lm[:, None], cols, jnp.int32(2**30)),
                     axis=1) + jj * _TS
        mins.append(lm)
        args.append(la)
    v0b = mins[0].astype(jnp.bfloat16).astype(jnp.float32)
    upd = mins[1] < v0b
    bidx = jnp.where(upd, args[1], args[0])
    bval = jnp.where(upd, mins[1], mins[0])
    inds_ref[0] = bidx.reshape(1, _TM)
    dmin_ref[0] = bval.reshape(1, _TM)


def _dist_argmin(zf, embt, zn3, en2, bl):
    nbm = bl // _TM
    return pl.pallas_call(
        _dist_argmin_body,
        grid=(nbm,),
        in_specs=[
            pl.BlockSpec((_TM, _E_DIM), lambda i: (i, 0)),
            pl.BlockSpec((_E_DIM, _N_E), lambda i: (0, 0)),
            pl.BlockSpec((1, _TM, 1), lambda i: (i, 0, 0)),
            pl.BlockSpec((1, _N_E), lambda i: (0, 0)),
        ],
        out_specs=[
            pl.BlockSpec((1, 1, _TM), lambda i: (i, 0, 0)),
            pl.BlockSpec((1, 1, _TM), lambda i: (i, 0, 0)),
        ],
        out_shape=[
            jax.ShapeDtypeStruct((nbm, 1, _TM), jnp.int32),
            jax.ShapeDtypeStruct((nbm, 1, _TM), jnp.float32),
        ],
    )(zf, embt, zn3, en2)


def _sc_gather(embedding, inds, bl):
    info = plsc.get_sparse_core_info()
    nw = info.num_cores * info.num_subcores
    b_per_w = bl // nw
    mesh = plsc.VectorSubcoreMesh(core_axis_name="c", subcore_axis_name="s")

    @functools.partial(
        pl.kernel,
        out_type=jax.ShapeDtypeStruct((bl, _E_DIM), jnp.float32),
        mesh=mesh,
        scratch_types=[
            pltpu.VMEM((b_per_w,), jnp.int32),
            pltpu.VMEM((b_per_w, _E_DIM), jnp.float32),
            pltpu.SemaphoreType.DMA,
        ],
    )
    def gather(emb_hbm, idx_hbm, out_hbm, idx_v, rows_v, sem):
        wid = lax.axis_index("s") * info.num_cores + lax.axis_index("c")
        base = wid * b_per_w
        pltpu.sync_copy(idx_hbm.at[pl.ds(base, b_per_w)], idx_v)
        pltpu.async_copy(emb_hbm.at[idx_v], rows_v, sem).wait()
        pltpu.sync_copy(rows_v, out_hbm.at[pl.ds(base, b_per_w)])

    return gather(embedding, inds)


def kernel(z, embedding):
    B, C, L = z.shape
    bl = B * L
    zp = jnp.transpose(z, (0, 2, 1))
    zf = zp.reshape(-1, C)
    zn = jnp.sum(zf ** 2, axis=1)
    en = jnp.sum(embedding ** 2, axis=1)
    embt = embedding.T

    inds3, dmin3 = _dist_argmin(
        zf, embt, zn.reshape(bl // _TM, _TM, 1), en.reshape(1, _N_E), bl)
    inds = inds3.reshape(-1)
    dmin = dmin3.reshape(-1)

    zq_rows = _sc_gather(embedding, inds, bl)

    m = jnp.sum(dmin) / (B * C * L)
    loss = _BETA * m + m

    zq = zq_rows.reshape(B, L, C)
    z_q = zp + (zq - zp)
    z_q = jnp.transpose(z_q, (0, 2, 1))
    return (z_q, loss, inds)


# speedup vs baseline: 1.0660x; 1.0344x over previous
import functools

import jax
import jax.numpy as jnp
from jax import lax
from jax.experimental import pallas as pl
from jax.experimental.pallas import tpu as pltpu
from jax.experimental.pallas import tpu_sc as plsc

_N_E = 8192
_E_DIM = 256
_BETA = 0.1

_TM = 512
_TS = 4096


def _dist_argmin_body(zf_ref, emb_ref, zn_ref, en_ref, inds_ref, dmin_ref):
    zf = zf_ref[...]
    zn = zn_ref[0]
    cols = lax.broadcasted_iota(jnp.int32, (_TM, _TS), 1)
    mins, args = [], []
    for jj in range(_N_E // _TS):
        s = lax.dot_general(
            zf, emb_ref[jj * _TS:(jj + 1) * _TS, :],
            (((1,), (1,)), ((), ())),
            preferred_element_type=jnp.float32)
        en = en_ref[:, jj * _TS:(jj + 1) * _TS]
        d = (zn + en) - 2.0 * s
        lm = jnp.min(d, axis=1)
        la = jnp.min(jnp.where(d == lm[:, None], cols, jnp.int32(2**30)),
                     axis=1) + jj * _TS
        mins.append(lm)
        args.append(la)
    v0b = mins[0].astype(jnp.bfloat16).astype(jnp.float32)
    upd = mins[1] < v0b
    bidx = jnp.where(upd, args[1], args[0])
    bval = jnp.where(upd, mins[1], mins[0])
    inds_ref[0] = bidx.reshape(1, _TM)
    dmin_ref[0] = bval.reshape(1, _TM)


def _dist_argmin(zf, emb, zn3, en2, bl):
    nbm = bl // _TM
    return pl.pallas_call(
        _dist_argmin_body,
        grid=(nbm,),
        in_specs=[
            pl.BlockSpec((_TM, _E_DIM), lambda i: (i, 0)),
            pl.BlockSpec((_N_E, _E_DIM), lambda i: (0, 0)),
            pl.BlockSpec((1, _TM, 1), lambda i: (i, 0, 0)),
            pl.BlockSpec((1, _N_E), lambda i: (0, 0)),
        ],
        out_specs=[
            pl.BlockSpec((1, 1, _TM), lambda i: (i, 0, 0)),
            pl.BlockSpec((1, 1, _TM), lambda i: (i, 0, 0)),
        ],
        out_shape=[
            jax.ShapeDtypeStruct((nbm, 1, _TM), jnp.int32),
            jax.ShapeDtypeStruct((nbm, 1, _TM), jnp.float32),
        ],
    )(zf, emb, zn3, en2)


def _sc_gather(embedding, inds, bl):
    info = plsc.get_sparse_core_info()
    nw = info.num_cores * info.num_subcores
    b_per_w = bl // nw
    mesh = plsc.VectorSubcoreMesh(core_axis_name="c", subcore_axis_name="s")

    @functools.partial(
        pl.kernel,
        out_type=jax.ShapeDtypeStruct((bl, _E_DIM), jnp.float32),
        mesh=mesh,
        scratch_types=[
            pltpu.VMEM((b_per_w,), jnp.int32),
            pltpu.VMEM((b_per_w, _E_DIM), jnp.float32),
            pltpu.SemaphoreType.DMA,
        ],
    )
    def gather(emb_hbm, idx_hbm, out_hbm, idx_v, rows_v, sem):
        wid = lax.axis_index("s") * info.num_cores + lax.axis_index("c")
        base = wid * b_per_w
        pltpu.sync_copy(idx_hbm.at[pl.ds(base, b_per_w)], idx_v)
        pltpu.async_copy(emb_hbm.at[idx_v], rows_v, sem).wait()
        pltpu.sync_copy(rows_v, out_hbm.at[pl.ds(base, b_per_w)])

    return gather(embedding, inds)


def kernel(z, embedding):
    B, C, L = z.shape
    bl = B * L
    zp = jnp.transpose(z, (0, 2, 1))
    zf = zp.reshape(-1, C)
    zn = jnp.sum(zf ** 2, axis=1)
    en = jnp.sum(embedding ** 2, axis=1)

    inds3, dmin3 = _dist_argmin(
        zf, embedding, zn.reshape(bl // _TM, _TM, 1), en.reshape(1, _N_E), bl)
    inds = inds3.reshape(-1)
    dmin = dmin3.reshape(-1)

    zq_rows = _sc_gather(embedding, inds, bl)

    m = jnp.sum(dmin) / (B * C * L)
    loss = _BETA * m + m

    zq = zq_rows.reshape(B, L, C)
    z_q = zp + (zq - zp)
    z_q = jnp.transpose(z_q, (0, 2, 1))
    return (z_q, loss, inds)


# speedup vs baseline: 1.0898x; 1.0223x over previous
import functools

import jax
import jax.numpy as jnp
from jax import lax
from jax.experimental import pallas as pl
from jax.experimental.pallas import tpu as pltpu
from jax.experimental.pallas import tpu_sc as plsc

_N_E = 8192
_E_DIM = 256
_BETA = 0.1

_TM = 512
_TS = 4096


def _dist_argmin_body(zt_ref, emb_ref, zn_ref, en_ref, inds_ref, dmin_ref):
    zt = zt_ref[0]
    zn = zn_ref[0]
    cols = lax.broadcasted_iota(jnp.int32, (_TM, _TS), 1)
    mins, args = [], []
    for jj in range(_N_E // _TS):
        s = lax.dot_general(
            zt, emb_ref[jj * _TS:(jj + 1) * _TS, :],
            (((0,), (1,)), ((), ())),
            preferred_element_type=jnp.float32)
        en = en_ref[:, jj * _TS:(jj + 1) * _TS]
        d = (zn + en) - 2.0 * s
        lm = jnp.min(d, axis=1)
        la = jnp.min(jnp.where(d == lm[:, None], cols, jnp.int32(2**30)),
                     axis=1) + jj * _TS
        mins.append(lm)
        args.append(la)
    v0b = mins[0].astype(jnp.bfloat16).astype(jnp.float32)
    upd = mins[1] < v0b
    bidx = jnp.where(upd, args[1], args[0])
    bval = jnp.where(upd, mins[1], mins[0])
    inds_ref[0] = bidx.reshape(1, _TM)
    dmin_ref[0] = bval.reshape(1, _TM)


def _dist_argmin(z, emb, zn3, en2, bl):
    nbm = bl // _TM
    lsplit = z.shape[2] // _TM
    return pl.pallas_call(
        _dist_argmin_body,
        grid=(nbm,),
        in_specs=[
            pl.BlockSpec((1, _E_DIM, _TM),
                         lambda i: (i // lsplit, 0, i % lsplit)),
            pl.BlockSpec((_N_E, _E_DIM), lambda i: (0, 0)),
            pl.BlockSpec((1, _TM, 1), lambda i: (i, 0, 0)),
            pl.BlockSpec((1, _N_E), lambda i: (0, 0)),
        ],
        out_specs=[
            pl.BlockSpec((1, 1, _TM), lambda i: (i, 0, 0)),
            pl.BlockSpec((1, 1, _TM), lambda i: (i, 0, 0)),
        ],
        out_shape=[
            jax.ShapeDtypeStruct((nbm, 1, _TM), jnp.int32),
            jax.ShapeDtypeStruct((nbm, 1, _TM), jnp.float32),
        ],
    )(z, emb, zn3, en2)


def _sc_gather(embedding, inds, bl):
    info = plsc.get_sparse_core_info()
    nw = info.num_cores * info.num_subcores
    b_per_w = bl // nw
    mesh = plsc.VectorSubcoreMesh(core_axis_name="c", subcore_axis_name="s")

    @functools.partial(
        pl.kernel,
        out_type=jax.ShapeDtypeStruct((bl, _E_DIM), jnp.float32),
        mesh=mesh,
        scratch_types=[
            pltpu.VMEM((b_per_w,), jnp.int32),
            pltpu.VMEM((b_per_w, _E_DIM), jnp.float32),
            pltpu.SemaphoreType.DMA,
        ],
    )
    def gather(emb_hbm, idx_hbm, out_hbm, idx_v, rows_v, sem):
        wid = lax.axis_index("s") * info.num_cores + lax.axis_index("c")
        base = wid * b_per_w
        pltpu.sync_copy(idx_hbm.at[pl.ds(base, b_per_w)], idx_v)
        pltpu.async_copy(emb_hbm.at[idx_v], rows_v, sem).wait()
        pltpu.sync_copy(rows_v, out_hbm.at[pl.ds(base, b_per_w)])

    return gather(embedding, inds)


def kernel(z, embedding):
    B, C, L = z.shape
    bl = B * L
    zp = jnp.transpose(z, (0, 2, 1))
    zf = zp.reshape(-1, C)
    zn = jnp.sum(zf ** 2, axis=1)
    en = jnp.sum(embedding ** 2, axis=1)

    inds3, dmin3 = _dist_argmin(
        z, embedding, zn.reshape(bl // _TM, _TM, 1), en.reshape(1, _N_E), bl)
    inds = inds3.reshape(-1)
    dmin = dmin3.reshape(-1)

    zq_rows = _sc_gather(embedding, inds, bl)

    m = jnp.sum(dmin) / (B * C * L)
    loss = _BETA * m + m

    zq = zq_rows.reshape(B, L, C)
    z_q = zp + (zq - zp)
    z_q = jnp.transpose(z_q, (0, 2, 1))
    return (z_q, loss, inds)


# speedup vs baseline: 1.1759x; 1.0791x over previous
import functools

import jax
import jax.numpy as jnp
from jax import lax
from jax.experimental import pallas as pl
from jax.experimental.pallas import tpu as pltpu
from jax.experimental.pallas import tpu_sc as plsc

_N_E = 8192
_E_DIM = 256
_BETA = 0.1

_TM = 512
_TS = 4096


def _dist_argmin_body(zt_ref, emb_ref, zn_ref, en_ref, inds_ref, dmin_ref):
    zt = zt_ref[0]
    zn = zn_ref[0]
    cols = lax.broadcasted_iota(jnp.int32, (_TM, _TS), 1)
    mins, args = [], []
    for jj in range(_N_E // _TS):
        s = lax.dot_general(
            zt, emb_ref[jj * _TS:(jj + 1) * _TS, :],
            (((0,), (1,)), ((), ())),
            preferred_element_type=jnp.float32)
        en = en_ref[:, jj * _TS:(jj + 1) * _TS]
        d = (zn + en) - 2.0 * s
        lm = jnp.min(d, axis=1)
        la = jnp.min(jnp.where(d == lm[:, None], cols, jnp.int32(2**30)),
                     axis=1) + jj * _TS
        mins.append(lm)
        args.append(la)
    v0b = mins[0].astype(jnp.bfloat16).astype(jnp.float32)
    upd = mins[1] < v0b
    bidx = jnp.where(upd, args[1], args[0])
    bval = jnp.where(upd, mins[1], mins[0])
    inds_ref[0] = bidx.reshape(1, _TM)
    dmin_ref[0] = bval.reshape(1, _TM)


def _dist_argmin(z, emb, zn3, en2, bl):
    nbm = bl // _TM
    lsplit = z.shape[2] // _TM
    return pl.pallas_call(
        _dist_argmin_body,
        grid=(nbm,),
        in_specs=[
            pl.BlockSpec((1, _E_DIM, _TM),
                         lambda i: (i // lsplit, 0, i % lsplit)),
            pl.BlockSpec((_N_E, _E_DIM), lambda i: (0, 0)),
            pl.BlockSpec((1, _TM, 1), lambda i: (i, 0, 0)),
            pl.BlockSpec((1, _N_E), lambda i: (0, 0)),
        ],
        out_specs=[
            pl.BlockSpec((1, 1, _TM), lambda i: (i, 0, 0)),
            pl.BlockSpec((1, 1, _TM), lambda i: (i, 0, 0)),
        ],
        out_shape=[
            jax.ShapeDtypeStruct((nbm, 1, _TM), jnp.int32),
            jax.ShapeDtypeStruct((nbm, 1, _TM), jnp.float32),
        ],
    )(z, emb, zn3, en2)


def _sc_gather(embedding, inds, bl):
    info = plsc.get_sparse_core_info()
    nw = info.num_cores * info.num_subcores
    b_per_w = bl // nw
    mesh = plsc.VectorSubcoreMesh(core_axis_name="c", subcore_axis_name="s")

    @functools.partial(
        pl.kernel,
        out_type=jax.ShapeDtypeStruct((bl, _E_DIM), jnp.float32),
        mesh=mesh,
        scratch_types=[
            pltpu.VMEM((b_per_w,), jnp.int32),
            pltpu.VMEM((b_per_w, _E_DIM), jnp.float32),
            pltpu.SemaphoreType.DMA,
        ],
    )
    def gather(emb_hbm, idx_hbm, out_hbm, idx_v, rows_v, sem):
        wid = lax.axis_index("s") * info.num_cores + lax.axis_index("c")
        base = wid * b_per_w
        pltpu.sync_copy(idx_hbm.at[pl.ds(base, b_per_w)], idx_v)
        pltpu.async_copy(emb_hbm.at[idx_v], rows_v, sem).wait()
        pltpu.sync_copy(rows_v, out_hbm.at[pl.ds(base, b_per_w)])

    return gather(embedding, inds)


def kernel(z, embedding):
    B, C, L = z.shape
    bl = B * L
    zp = jnp.transpose(z, (0, 2, 1))
    zf = zp.reshape(-1, C)
    zn = jnp.sum(zf ** 2, axis=1)
    en = jnp.sum(embedding ** 2, axis=1)

    inds3, dmin3 = _dist_argmin(
        z, embedding, zn.reshape(bl // _TM, _TM, 1), en.reshape(1, _N_E), bl)
    inds = inds3.reshape(-1)
    dmin = dmin3.reshape(-1)

    zq_rows = _sc_gather(embedding, inds, bl)

    m = jnp.sum(dmin) / (B * C * L)
    loss = _BETA * m + m

    zq = zq_rows.reshape(B, L, C)
    z_q = jnp.transpose(zq, (0, 2, 1))
    return (z_q, loss, inds)
